# baseline (speedup 1.0000x reference)
import functools

import jax
import jax.numpy as jnp
from jax import lax
from jax.experimental import pallas as pl
from jax.experimental.pallas import tpu as pltpu
from jax.experimental.pallas import tpu_sc as plsc

_BATCH = 16384
_DIM = 64
_NW = 32
_BPW = _BATCH // _NW
_IDX_CHUNK = 128
_NCHUNK = _BPW // _IDX_CHUNK
_LANES = 16


def _cbow_body(x0_hbm, x1_hbm, x2_hbm, a_hbm, b_hbm, out_hbm, idx_v, rows0,
               rows1, rows2, out_v, sem):
    wid = lax.axis_index("s") * 2 + lax.axis_index("c")
    base = wid * _BPW

    for t, col_hbm in enumerate((x0_hbm, x1_hbm, x2_hbm)):
        for j in range(_NCHUNK):
            pltpu.sync_copy(
                col_hbm.at[pl.ds(base + j * _IDX_CHUNK, _IDX_CHUNK)],
                idx_v.at[t * _NCHUNK + j],
            )

    copies = []
    for t, rows in enumerate((rows0, rows1, rows2)):
        for j in range(_NCHUNK):
            copies.append(
                pltpu.async_copy(
                    (a_hbm if t < 2 else b_hbm).at[idx_v.at[t * _NCHUNK + j]],
                    rows.at[pl.ds(j * _IDX_CHUNK, _IDX_CHUNK)],
                    sem,
                )
            )
    for c in copies:
        c.wait()

    lane_iota = lax.iota(jnp.int32, _LANES)

    def group_body(g, _):
        r = g * _LANES
        row_ids = r + lane_iota
        acc = jnp.zeros((_LANES,), jnp.float32)
        for d in range(_DIM):
            col = jnp.full((_LANES,), d, jnp.int32)
            a0 = plsc.load_gather(rows0, [row_ids, col])
            a1 = plsc.load_gather(rows1, [row_ids, col])
            bv = plsc.load_gather(rows2, [row_ids, col])
            acc = acc + (a0 + a1) * bv
        out_v[pl.ds(r, _LANES)] = acc * 0.5
        return 0

    lax.fori_loop(0, _BPW // _LANES, group_body, 0)

    pltpu.sync_copy(out_v, out_hbm.at[pl.ds(base, _BPW)])


@jax.jit
def _cbow(x0, x1, x2, A, B):
    mesh = plsc.VectorSubcoreMesh(core_axis_name="c", subcore_axis_name="s")
    f = pl.kernel(
        _cbow_body,
        out_type=jax.ShapeDtypeStruct((_BATCH,), jnp.float32),
        mesh=mesh,
        scratch_types=[
            pltpu.VMEM((3 * _NCHUNK, _IDX_CHUNK), jnp.int32),
            pltpu.VMEM((_BPW, _DIM), jnp.float32),
            pltpu.VMEM((_BPW, _DIM), jnp.float32),
            pltpu.VMEM((_BPW, _DIM), jnp.float32),
            pltpu.VMEM((_BPW,), jnp.float32),
            pltpu.SemaphoreType.DMA,
        ],
        compiler_params=pltpu.CompilerParams(
            needs_layout_passes=False, use_tc_tiling_on_sc=False
        ),
    )
    return f(x0, x1, x2, A, B)


def kernel(x, A, B):
    xi = x.astype(jnp.int32)
    return _cbow(xi[:, 0], xi[:, 1], xi[:, 2], A, B)
